# Initial kernel scaffold; baseline (speedup 1.0000x reference)
import functools

import jax
import jax.numpy as jnp
from jax import lax
from jax.experimental import pallas as pl
from jax.experimental.pallas import tpu as pltpu
from jax.experimental.pallas import tpu_sc as plsc

N = 10000
NPAD = 10240
E = 160000
EP = 163840
DIN = 256
DOUT = 128
AUG = 144
DUMMY = N

_MESH = dict(core_axis_name="c", subcore_axis_name="s", num_cores=2,
             num_subcores=16)

F32 = jnp.float32
I32 = jnp.int32


def _deg_body(dst_hbm, w_hbm, out_hbm, tab, dstv, wv, rows, sem):
    del sem
    cid = lax.axis_index("c")
    sid = lax.axis_index("s")

    def zero(i, c):
        rows[i, :] = jnp.zeros((16,), F32)
        return c
    lax.fori_loop(0, 1024, zero, 0)
    pltpu.sync_copy(rows.at[pl.ds(0, 512), :], tab.at[pl.ds(sid * 640, 512), :])
    pltpu.sync_copy(rows.at[pl.ds(0, 128), :],
                    tab.at[pl.ds(sid * 640 + 512, 128), :])
    plsc.subcore_barrier()

    base_row = cid * (EP // 2 // 128) + sid * (EP // 32 // 128)

    def chunk(k, c):
        roff = base_row + k * 8
        pltpu.sync_copy(dst_hbm.at[pl.ds(roff, 8), :], dstv)
        pltpu.sync_copy(w_hbm.at[pl.ds(roff, 8), :], wv)

        def put(i, c2):
            w16 = wv[i // 8, pl.ds((i % 8) * 16, 16)]
            ridx = lax.broadcasted_iota(I32, (16,), 0) + i * 16
            plsc.store_scatter(rows, [ridx, jnp.zeros((16,), I32)], w16)
            return c2
        lax.fori_loop(0, 64, put, 0)
        for j in range(8):
            pltpu.sync_copy(rows.at[pl.ds(j * 128, 128), :],
                            tab.at[dstv.at[j]], add=True)
        return c
    lax.fori_loop(0, 5, chunk, 0)
    plsc.subcore_barrier()
    pltpu.sync_copy(tab.at[pl.ds(sid * 640, 640), :],
                    out_hbm.at[cid, pl.ds(sid * 640, 640), :])


def _sc_deg(dst2, w2):
    f = pl.kernel(
        _deg_body,
        out_type=jax.ShapeDtypeStruct((2, NPAD, 16), F32),
        mesh=plsc.VectorSubcoreMesh(**_MESH),
        scratch_types=[
            pltpu.VMEM_SHARED((NPAD, 16), F32),
            pltpu.VMEM((8, 128), I32),
            pltpu.VMEM((8, 128), F32),
            pltpu.VMEM((1024, 16), F32),
            pltpu.SemaphoreType.DMA,
        ],
    )
    return f(dst2, w2)


def _acc1_body(srcb_hbm, dst_hbm, w_hbm, g2_hbm, out_hbm,
               tab, srcv, dstv, wv, rows, sem):
    cid = lax.axis_index("c")
    sid = lax.axis_index("s")

    def zero(i, c):
        for j in range(8):
            rows[i, pl.ds(j * 16, 16)] = jnp.zeros((16,), F32)
        return c
    lax.fori_loop(0, 512, zero, 0)
    pltpu.sync_copy(rows.at[pl.ds(0, 512), :], tab.at[pl.ds(sid * 640, 512), :])
    pltpu.sync_copy(rows.at[pl.ds(0, 128), :],
                    tab.at[pl.ds(sid * 640 + 512, 128), :])
    plsc.subcore_barrier()

    base_row = sid * (EP // 16 // 128)

    def chunk(k, c):
        roff = base_row + k * 4
        pltpu.sync_copy(srcb_hbm.at[pl.ds(cid * (EP // 128) + roff, 4), :],
                        srcv)
        pltpu.sync_copy(dst_hbm.at[pl.ds(roff, 4), :], dstv)
        pltpu.sync_copy(w_hbm.at[pl.ds(roff, 4), :], wv)
        cps = [pltpu.async_copy(g2_hbm.at[srcv.at[j]],
                                rows.at[pl.ds(j * 128, 128), :], sem)
               for j in range(4)]
        for cp in cps:
            cp.wait()

        def scale(e, c2):
            ws = wv[e // 128, e % 128]
            for j in range(8):
                sl = pl.ds(j * 16, 16)
                rows[e, sl] = rows[e, sl] * ws
            return c2
        lax.fori_loop(0, 512, scale, 0)
        for j in range(4):
            pltpu.sync_copy(rows.at[pl.ds(j * 128, 128), :],
                            tab.at[dstv.at[j]], add=True)
        return c
    lax.fori_loop(0, 20, chunk, 0)
    plsc.subcore_barrier()
    pltpu.sync_copy(tab.at[pl.ds(sid * 640, 640), :],
                    out_hbm.at[cid, pl.ds(sid * 640, 640), :])


def _sc_acc1(srcb2, dst2, w2, g2):
    f = pl.kernel(
        _acc1_body,
        out_type=jax.ShapeDtypeStruct((2, NPAD, 128), F32),
        mesh=plsc.VectorSubcoreMesh(**_MESH),
        scratch_types=[
            pltpu.VMEM_SHARED((NPAD, 128), F32),
            pltpu.VMEM((4, 128), I32),
            pltpu.VMEM((4, 128), I32),
            pltpu.VMEM((4, 128), F32),
            pltpu.VMEM((512, 128), F32),
            pltpu.SemaphoreType.DMA,
        ],
    )
    return f(srcb2, dst2, w2, g2)


def _acc2_body(src_hbm, dst_hbm, w_hbm, hga_hbm, asd_hbm, c_hbm, out_hbm,
               tab, srcv, dstv, wv, pv, rows, asdv, cbuf, sem):
    cid = lax.axis_index("c")
    sid = lax.axis_index("s")

    def zero(i, c):
        for j in range(9):
            rows[i, pl.ds(j * 16, 16)] = jnp.zeros((16,), F32)
        return c
    lax.fori_loop(0, 512, zero, 0)
    pltpu.sync_copy(rows.at[pl.ds(0, 512), :], tab.at[pl.ds(sid * 640, 512), :])
    pltpu.sync_copy(rows.at[pl.ds(0, 128), :],
                    tab.at[pl.ds(sid * 640 + 512, 128), :])
    pltpu.sync_copy(asd_hbm, asdv)
    pltpu.sync_copy(c_hbm, cbuf)
    plsc.subcore_barrier()

    cval = cbuf[0, 0]
    base_row = cid * (EP // 2 // 128) + sid * (EP // 32 // 128)

    def chunk(k, c):
        roff = base_row + k * 4
        pltpu.sync_copy(src_hbm.at[pl.ds(roff, 4), :], srcv)
        pltpu.sync_copy(dst_hbm.at[pl.ds(roff, 4), :], dstv)
        pltpu.sync_copy(w_hbm.at[pl.ds(roff, 4), :], wv)
        cps = [pltpu.async_copy(hga_hbm.at[srcv.at[j]],
                                rows.at[pl.ds(j * 128, 128), :], sem)
               for j in range(4)]

        zc = jnp.zeros((16,), I32)
        oc = jnp.ones((16,), I32)

        def pcomp(i, c2):
            sl = pl.ds((i % 8) * 16, 16)
            s16 = srcv[i // 8, sl]
            d16 = dstv[i // 8, sl]
            w16 = wv[i // 8, sl]
            a_s = plsc.load_gather(asdv, [s16, zc])
            a_d = plsc.load_gather(asdv, [d16, oc])
            l = a_s + a_d + cval * w16
            l = jnp.where(l >= 0, l, 0.2 * l)
            pv[i // 8, sl] = jnp.exp(l)
            return c2
        lax.fori_loop(0, 32, pcomp, 0)
        for cp in cps:
            cp.wait()

        def scale(e, c2):
            ps = pv[e // 128, e % 128]
            for j in range(9):
                sl = pl.ds(j * 16, 16)
                rows[e, sl] = rows[e, sl] * ps
            return c2
        lax.fori_loop(0, 512, scale, 0)
        for j in range(4):
            pltpu.sync_copy(rows.at[pl.ds(j * 128, 128), :],
                            tab.at[dstv.at[j]], add=True)
        return c
    lax.fori_loop(0, 10, chunk, 0)
    plsc.subcore_barrier()
    pltpu.sync_copy(tab.at[pl.ds(sid * 640, 640), :],
                    out_hbm.at[cid, pl.ds(sid * 640, 640), :])


def _sc_acc2(src2, dst2, w2, hga, asd, cpad):
    f = pl.kernel(
        _acc2_body,
        out_type=jax.ShapeDtypeStruct((2, NPAD, AUG), F32),
        mesh=plsc.VectorSubcoreMesh(**_MESH),
        scratch_types=[
            pltpu.VMEM_SHARED((NPAD, AUG), F32),
            pltpu.VMEM((4, 128), I32),
            pltpu.VMEM((4, 128), I32),
            pltpu.VMEM((4, 128), F32),
            pltpu.VMEM((4, 128), F32),
            pltpu.VMEM((512, AUG), F32),
            pltpu.VMEM((NPAD, 2), F32),
            pltpu.VMEM((8, 128), F32),
            pltpu.SemaphoreType.DMA,
        ],
    )
    return f(src2, dst2, w2, hga, asd, cpad)


def _tc1_body(emb_ref, w1_ref, deg_ref, out_ref):
    d = deg_ref[...]
    deg = d[0, :, 0] + d[1, :, 0] + 1.0
    dinv = lax.rsqrt(deg)
    h = jnp.dot(emb_ref[...], w1_ref[...], preferred_element_type=F32)
    g = h * dinv[:, None]
    out_ref[0] = g[:, :128]
    out_ref[1] = g[:, 128:]


def _tc1(emb_pad, W1, deg_tab):
    return pl.pallas_call(
        _tc1_body,
        grid=(20,),
        in_specs=[
            pl.BlockSpec((512, DIN), lambda i: (i, 0)),
            pl.BlockSpec((DIN, DIN), lambda i: (0, 0)),
            pl.BlockSpec((2, 512, 16), lambda i: (0, i, 0)),
        ],
        out_specs=pl.BlockSpec((2, 512, 128), lambda i: (0, i, 0)),
        out_shape=jax.ShapeDtypeStruct((2, NPAD, 128), F32),
    )(emb_pad, W1, deg_tab)


def _tc2_body(acc_ref, g_ref, deg_ref, b1_ref, w2_ref, as_ref, ad_ref,
              lin_ref, ae_ref, hga_ref, asd_ref, c_ref):
    acc = acc_ref[...]
    g = g_ref[...]
    afull = jnp.concatenate([acc[0], acc[1]], axis=1)
    gfull = jnp.concatenate([g[0], g[1]], axis=1)
    d = deg_ref[...]
    deg = d[0, :, 0] + d[1, :, 0] + 1.0
    dinv = lax.rsqrt(deg)
    out1 = (afull + gfull) * dinv[:, None] + b1_ref[...]
    x2 = jnp.maximum(out1, 0.0)
    hg = jnp.dot(x2, w2_ref[...], preferred_element_type=F32)
    hga_ref[...] = jnp.concatenate(
        [hg, jnp.ones((512, 1), F32), jnp.zeros((512, AUG - 129), F32)],
        axis=1)
    att2 = jnp.concatenate([as_ref[...], ad_ref[...]], axis=0)
    asd_ref[...] = lax.dot_general(hg, att2, (((1,), (1,)), ((), ())),
                                   preferred_element_type=F32)
    cv = jnp.sum(lin_ref[...] * ae_ref[...])
    c_ref[...] = jnp.full((8, 128), cv, F32)


def _tc2(acc1, g2v, deg_tab, b1, W2, att_src, att_dst, lin_edge, att_edge):
    return pl.pallas_call(
        _tc2_body,
        grid=(20,),
        in_specs=[
            pl.BlockSpec((2, 512, 128), lambda i: (0, i, 0)),
            pl.BlockSpec((2, 512, 128), lambda i: (0, i, 0)),
            pl.BlockSpec((2, 512, 16), lambda i: (0, i, 0)),
            pl.BlockSpec((1, DIN), lambda i: (0, 0)),
            pl.BlockSpec((DIN, DOUT), lambda i: (0, 0)),
            pl.BlockSpec((1, DOUT), lambda i: (0, 0)),
            pl.BlockSpec((1, DOUT), lambda i: (0, 0)),
            pl.BlockSpec((1, DOUT), lambda i: (0, 0)),
            pl.BlockSpec((1, DOUT), lambda i: (0, 0)),
        ],
        out_specs=[
            pl.BlockSpec((512, AUG), lambda i: (i, 0)),
            pl.BlockSpec((512, 2), lambda i: (i, 0)),
            pl.BlockSpec((8, 128), lambda i: (0, 0)),
        ],
        out_shape=[
            jax.ShapeDtypeStruct((NPAD, AUG), F32),
            jax.ShapeDtypeStruct((NPAD, 2), F32),
            jax.ShapeDtypeStruct((8, 128), F32),
        ],
    )(acc1, g2v, deg_tab, b1, W2, att_src, att_dst, lin_edge, att_edge)


def _tc3_body(acc2_ref, hga_ref, asd_ref, c_ref, b2_ref, out_ref):
    a = acc2_ref[...]
    S = a[0, :N, :128] + a[1, :N, :128]
    den_r = a[0, :N, 128] + a[1, :N, 128]
    hg = hga_ref[...][:N, :128]
    asd = asd_ref[...]
    cv = c_ref[0, 0]
    ll = asd[:N, 0] + asd[:N, 1] + cv
    ll = jnp.where(ll >= 0, ll, 0.2 * ll)
    p_loop = jnp.exp(ll)
    num = S + p_loop[:, None] * hg
    den = den_r + p_loop + 1e-16
    out2 = num / den[:, None] + b2_ref[...]
    m = jnp.max(out2, axis=0)
    ex = jnp.exp(out2 - m[None, :])
    ssum = jnp.sum(ex, axis=0)
    out_ref[...] = ex / ssum[None, :]


def _tc3(acc2, hga, asd, cpad, b2):
    return pl.pallas_call(
        _tc3_body,
        in_specs=[
            pl.BlockSpec((2, NPAD, AUG), lambda: (0, 0, 0)),
            pl.BlockSpec((NPAD, AUG), lambda: (0, 0)),
            pl.BlockSpec((NPAD, 2), lambda: (0, 0)),
            pl.BlockSpec((8, 128), lambda: (0, 0)),
            pl.BlockSpec((1, DOUT), lambda: (0, 0)),
        ],
        out_specs=pl.BlockSpec((N, DOUT), lambda: (0, 0)),
        out_shape=jax.ShapeDtypeStruct((N, DOUT), F32),
    )(acc2, hga, asd, cpad, b2)


def kernel(nodes, edge_index, edge_weight, emb, W1, b1, W2, att_src, att_dst,
           lin_edge, att_edge, b2):
    del nodes
    src = edge_index[0].astype(I32)
    dst = edge_index[1].astype(I32)
    w = edge_weight.astype(F32)

    padn = EP - E
    srcp = jnp.concatenate([src, jnp.zeros((padn,), I32)])
    dstp = jnp.concatenate([dst, jnp.full((padn,), DUMMY, I32)])
    wp = jnp.concatenate([w, jnp.zeros((padn,), F32)])
    src2 = srcp.reshape(EP // 128, 128)
    dst2 = dstp.reshape(EP // 128, 128)
    w2 = wp.reshape(EP // 128, 128)
    srcb2 = jnp.concatenate([srcp, srcp + NPAD]).reshape(2 * EP // 128, 128)

    emb_pad = jnp.pad(emb, ((0, NPAD - N), (0, 0)))

    deg_tab = _sc_deg(dst2, w2)
    g2v = _tc1(emb_pad, W1, deg_tab)
    g2flat = g2v.reshape(2 * NPAD, 128)
    acc1 = _sc_acc1(srcb2, dst2, w2, g2flat)
    hga, asd, cpad = _tc2(acc1, g2v, deg_tab, b1[None, :], W2,
                          att_src[None, :], att_dst[None, :],
                          lin_edge, att_edge[None, :])
    acc2 = _sc_acc2(src2, dst2, w2, hga, asd, cpad)
    return _tc3(acc2, hga, asd, cpad, b2[None, :])


# speedup vs baseline: 7.0449x; 7.0449x over previous
import functools

import jax
import jax.numpy as jnp
from jax import lax
from jax.experimental import pallas as pl
from jax.experimental.pallas import tpu as pltpu
from jax.experimental.pallas import tpu_sc as plsc

N = 10000
NPAD = 10240
E = 160000
EP = 163840
DIN = 256
DOUT = 128
AUG = 80
DUMMY = N

_MESH = dict(core_axis_name="c", subcore_axis_name="s", num_cores=2,
             num_subcores=16)

F32 = jnp.float32
I32 = jnp.int32


def _deg_body(dst_hbm, w_hbm, out_hbm, tab, dstv, wv, rows, sem):
    del sem
    cid = lax.axis_index("c")
    sid = lax.axis_index("s")

    def zero(i, c):
        rows[i, :] = jnp.zeros((16,), F32)
        return c
    lax.fori_loop(0, 1024, zero, 0)
    pltpu.sync_copy(rows.at[pl.ds(0, 512), :], tab.at[pl.ds(sid * 640, 512), :])
    pltpu.sync_copy(rows.at[pl.ds(0, 128), :],
                    tab.at[pl.ds(sid * 640 + 512, 128), :])
    plsc.subcore_barrier()

    base_row = cid * (EP // 2 // 128) + sid * (EP // 32 // 128)
    lane0 = jnp.where(lax.broadcasted_iota(I32, (16,), 0) == 0,
                      jnp.float32(1.0), jnp.float32(0.0))

    def chunk(k, c):
        roff = base_row + k * 8
        pltpu.sync_copy(dst_hbm.at[pl.ds(roff, 8), :], dstv)
        for j in range(8):
            pltpu.sync_copy(w_hbm.at[roff + j], wv.at[pl.ds(j * 128, 128)])

        def put(e, c2):
            ws = wv[pl.ds(e, 16)][0]
            rows[e, :] = lane0 * ws
            return c2
        lax.fori_loop(0, 1024, put, 0)
        for j in range(8):
            pltpu.sync_copy(rows.at[pl.ds(j * 128, 128), :],
                            tab.at[dstv.at[j]], add=True)
        return c
    lax.fori_loop(0, 5, chunk, 0)
    plsc.subcore_barrier()
    pltpu.sync_copy(tab.at[pl.ds(sid * 640, 640), :],
                    out_hbm.at[cid, pl.ds(sid * 640, 640), :])


def _sc_deg(dst2, w2):
    f = pl.kernel(
        _deg_body,
        out_type=jax.ShapeDtypeStruct((2, NPAD, 16), F32),
        mesh=plsc.VectorSubcoreMesh(**_MESH),
        compiler_params=pltpu.CompilerParams(use_tc_tiling_on_sc=False),
        scratch_types=[
            pltpu.VMEM_SHARED((NPAD, 16), F32),
            pltpu.VMEM((8, 128), I32),
            pltpu.VMEM((1040,), F32),
            pltpu.VMEM((1024, 16), F32),
            pltpu.SemaphoreType.DMA,
        ],
    )
    return f(dst2, w2)


def _acc1_body(srcb_hbm, dst_hbm, w_hbm, g2_hbm, out_hbm,
               tab, srcv, dstv, wv, rows, sem):
    cid = lax.axis_index("c")
    sid = lax.axis_index("s")

    def zero(i, c):
        for j in range(8):
            rows[i, pl.ds(j * 16, 16)] = jnp.zeros((16,), F32)
        return c
    lax.fori_loop(0, 256, zero, 0)
    for q in range(2):
        pltpu.sync_copy(rows, tab.at[pl.ds(sid * 640 + q * 256, 256), :])
    pltpu.sync_copy(rows.at[pl.ds(0, 128), :],
                    tab.at[pl.ds(sid * 640 + 512, 128), :])
    plsc.subcore_barrier()

    base_row = sid * (EP // 16 // 128)

    def chunk(k, c):
        roff = base_row + k * 2
        pltpu.sync_copy(srcb_hbm.at[pl.ds(cid * (EP // 128) + roff, 2), :],
                        srcv)
        pltpu.sync_copy(dst_hbm.at[pl.ds(roff, 2), :], dstv)
        for j in range(2):
            pltpu.sync_copy(w_hbm.at[roff + j], wv.at[pl.ds(j * 128, 128)])
        cps = [pltpu.async_copy(g2_hbm.at[srcv.at[j]],
                                rows.at[pl.ds(j * 128, 128), :], sem)
               for j in range(2)]
        for cp in cps:
            cp.wait()

        def scale(e, c2):
            ws = wv[pl.ds(e, 16)][0]
            for j in range(8):
                sl = pl.ds(j * 16, 16)
                rows[e, sl] = rows[e, sl] * ws
            return c2
        lax.fori_loop(0, 256, scale, 0)
        for j in range(2):
            pltpu.sync_copy(rows.at[pl.ds(j * 128, 128), :],
                            tab.at[dstv.at[j]], add=True)
        return c
    lax.fori_loop(0, 40, chunk, 0)
    plsc.subcore_barrier()
    pltpu.sync_copy(tab.at[pl.ds(sid * 640, 640), :],
                    out_hbm.at[cid, pl.ds(sid * 640, 640), :])


def _sc_acc1(srcb2, dst2, w2, g2):
    f = pl.kernel(
        _acc1_body,
        out_type=jax.ShapeDtypeStruct((2, NPAD, 128), F32),
        mesh=plsc.VectorSubcoreMesh(**_MESH),
        compiler_params=pltpu.CompilerParams(use_tc_tiling_on_sc=False),
        scratch_types=[
            pltpu.VMEM_SHARED((NPAD, 128), F32),
            pltpu.VMEM((2, 128), I32),
            pltpu.VMEM((2, 128), I32),
            pltpu.VMEM((272,), F32),
            pltpu.VMEM((256, 128), F32),
            pltpu.SemaphoreType.DMA,
        ],
    )
    return f(srcb2, dst2, w2, g2)


def _acc2_body(srcb_hbm, dst_hbm, w_hbm, hgab_hbm, ad16_hbm, c_hbm, out_hbm,
               tab, srcv, dstv, wv, rows, adbuf, cbuf, sem):
    cid = lax.axis_index("c")
    sid = lax.axis_index("s")

    def zero(i, c):
        for j in range(5):
            rows[i, pl.ds(j * 16, 16)] = jnp.zeros((16,), F32)
        return c
    lax.fori_loop(0, 512, zero, 0)
    pltpu.sync_copy(rows.at[pl.ds(0, 512), :], tab.at[pl.ds(sid * 640, 512), :])
    pltpu.sync_copy(rows.at[pl.ds(0, 128), :],
                    tab.at[pl.ds(sid * 640 + 512, 128), :])
    pltpu.sync_copy(c_hbm, cbuf)
    plsc.subcore_barrier()

    cval = cbuf[0, pl.ds(0, 16)][0]
    base_row = sid * (EP // 16 // 128)

    def chunk(k, c):
        roff = base_row + k * 4
        pltpu.sync_copy(srcb_hbm.at[pl.ds(cid * (EP // 128) + roff, 4), :],
                        srcv)
        pltpu.sync_copy(dst_hbm.at[pl.ds(roff, 4), :], dstv)
        for j in range(4):
            pltpu.sync_copy(w_hbm.at[roff + j], wv.at[pl.ds(j * 128, 128)])
        cps = [pltpu.async_copy(hgab_hbm.at[srcv.at[j]],
                                rows.at[pl.ds(j * 128, 128), :], sem)
               for j in range(4)]
        cpa = [pltpu.async_copy(ad16_hbm.at[dstv.at[j]],
                                adbuf.at[pl.ds(j * 128, 128), :], sem)
               for j in range(4)]
        for cp in cps + cpa:
            cp.wait()

        def scale(e, c2):
            tail = rows[e, pl.ds(64, 16)]
            adrow = adbuf[e, :]
            wse = wv[pl.ds(e, 16)][0]
            l = tail[1] + adrow[0] + cval * wse
            l = jnp.where(l >= 0, l, 0.2 * l)
            pe = jnp.exp(jnp.full((16,), l, F32))
            for j in range(5):
                sl = pl.ds(j * 16, 16)
                rows[e, sl] = rows[e, sl] * pe
            return c2
        lax.fori_loop(0, 512, scale, 0)
        for j in range(4):
            pltpu.sync_copy(rows.at[pl.ds(j * 128, 128), :],
                            tab.at[dstv.at[j]], add=True)
        return c
    lax.fori_loop(0, 20, chunk, 0)
    plsc.subcore_barrier()
    pltpu.sync_copy(tab.at[pl.ds(sid * 640, 640), :],
                    out_hbm.at[cid, pl.ds(sid * 640, 640), :])


def _sc_acc2(srcb2, dst2, w2, hgab, ad16, cpad):
    f = pl.kernel(
        _acc2_body,
        out_type=jax.ShapeDtypeStruct((2, NPAD, AUG), F32),
        mesh=plsc.VectorSubcoreMesh(**_MESH),
        compiler_params=pltpu.CompilerParams(use_tc_tiling_on_sc=False),
        scratch_types=[
            pltpu.VMEM_SHARED((NPAD, AUG), F32),
            pltpu.VMEM((4, 128), I32),
            pltpu.VMEM((4, 128), I32),
            pltpu.VMEM((528,), F32),
            pltpu.VMEM((512, AUG), F32),
            pltpu.VMEM((512, 16), F32),
            pltpu.VMEM((8, 128), F32),
            pltpu.SemaphoreType.DMA,
        ],
    )
    return f(srcb2, dst2, w2, hgab, ad16, cpad)


def _tc1_body(emb_ref, w1_ref, deg_ref, out_ref):
    d = deg_ref[...]
    deg = d[0, :, 0] + d[1, :, 0] + 1.0
    dinv = lax.rsqrt(deg)
    h = jnp.dot(emb_ref[...], w1_ref[...], preferred_element_type=F32)
    g = h * dinv[:, None]
    out_ref[0] = g[:, :128]
    out_ref[1] = g[:, 128:]


def _tc1(emb_pad, W1, deg_tab):
    return pl.pallas_call(
        _tc1_body,
        grid=(20,),
        in_specs=[
            pl.BlockSpec((512, DIN), lambda i: (i, 0)),
            pl.BlockSpec((DIN, DIN), lambda i: (0, 0)),
            pl.BlockSpec((2, 512, 16), lambda i: (0, i, 0)),
        ],
        out_specs=pl.BlockSpec((2, 512, 128), lambda i: (0, i, 0)),
        out_shape=jax.ShapeDtypeStruct((2, NPAD, 128), F32),
    )(emb_pad, W1, deg_tab)


def _tc2_body(acc_ref, g_ref, deg_ref, b1_ref, w2_ref, as_ref, ad_ref,
              lin_ref, ae_ref, hga0_ref, hga1_ref, ad16_ref, c_ref):
    acc = acc_ref[...]
    g = g_ref[...]
    afull = jnp.concatenate([acc[0], acc[1]], axis=1)
    gfull = jnp.concatenate([g[0], g[1]], axis=1)
    d = deg_ref[...]
    deg = d[0, :, 0] + d[1, :, 0] + 1.0
    dinv = lax.rsqrt(deg)
    out1 = (afull + gfull) * dinv[:, None] + b1_ref[...]
    x2 = jnp.maximum(out1, 0.0)
    hg = jnp.dot(x2, w2_ref[...], preferred_element_type=F32)
    a_sc = lax.dot_general(hg, as_ref[...], (((1,), (1,)), ((), ())),
                           preferred_element_type=F32)
    a_dc = lax.dot_general(hg, ad_ref[...], (((1,), (1,)), ((), ())),
                           preferred_element_type=F32)
    ones = jnp.ones((512, 1), F32)
    zpad = jnp.zeros((512, AUG - 66), F32)
    hga0_ref[...] = jnp.concatenate([hg[:, :64], ones, a_sc, zpad], axis=1)
    hga1_ref[...] = jnp.concatenate([hg[:, 64:], ones, a_sc, zpad], axis=1)
    ad16_ref[...] = jnp.concatenate([a_dc, jnp.zeros((512, 15), F32)], axis=1)
    cv = jnp.sum(lin_ref[...] * ae_ref[...])
    c_ref[...] = jnp.full((8, 128), cv, F32)


def _tc2(acc1, g2v, deg_tab, b1, W2, att_src, att_dst, lin_edge, att_edge):
    return pl.pallas_call(
        _tc2_body,
        grid=(20,),
        in_specs=[
            pl.BlockSpec((2, 512, 128), lambda i: (0, i, 0)),
            pl.BlockSpec((2, 512, 128), lambda i: (0, i, 0)),
            pl.BlockSpec((2, 512, 16), lambda i: (0, i, 0)),
            pl.BlockSpec((1, DIN), lambda i: (0, 0)),
            pl.BlockSpec((DIN, DOUT), lambda i: (0, 0)),
            pl.BlockSpec((1, DOUT), lambda i: (0, 0)),
            pl.BlockSpec((1, DOUT), lambda i: (0, 0)),
            pl.BlockSpec((1, DOUT), lambda i: (0, 0)),
            pl.BlockSpec((1, DOUT), lambda i: (0, 0)),
        ],
        out_specs=[
            pl.BlockSpec((512, AUG), lambda i: (i, 0)),
            pl.BlockSpec((512, AUG), lambda i: (i, 0)),
            pl.BlockSpec((512, 16), lambda i: (i, 0)),
            pl.BlockSpec((8, 128), lambda i: (0, 0)),
        ],
        out_shape=[
            jax.ShapeDtypeStruct((NPAD, AUG), F32),
            jax.ShapeDtypeStruct((NPAD, AUG), F32),
            jax.ShapeDtypeStruct((NPAD, 16), F32),
            jax.ShapeDtypeStruct((8, 128), F32),
        ],
    )(acc1, g2v, deg_tab, b1, W2, att_src, att_dst, lin_edge, att_edge)


def _tc3_body(acc2_ref, hga0_ref, hga1_ref, ad16_ref, c_ref, b2_ref, out_ref):
    a = acc2_ref[...]
    S = jnp.concatenate([a[0, :N, :64], a[1, :N, :64]], axis=1)
    den_r = a[0, :N, 64]
    h0 = hga0_ref[...]
    h1 = hga1_ref[...]
    hg = jnp.concatenate([h0[:N, :64], h1[:N, :64]], axis=1)
    cv = c_ref[0, 0]
    ll = h0[:N, 65] + ad16_ref[...][:N, 0] + cv
    ll = jnp.where(ll >= 0, ll, 0.2 * ll)
    p_loop = jnp.exp(ll)
    num = S + p_loop[:, None] * hg
    den = den_r + p_loop + 1e-16
    out2 = num / den[:, None] + b2_ref[...]
    m = jnp.max(out2, axis=0)
    ex = jnp.exp(out2 - m[None, :])
    ssum = jnp.sum(ex, axis=0)
    out_ref[...] = ex / ssum[None, :]


def _tc3(acc2, hga0, hga1, ad16, cpad, b2):
    return pl.pallas_call(
        _tc3_body,
        in_specs=[
            pl.BlockSpec((2, NPAD, AUG), lambda: (0, 0, 0)),
            pl.BlockSpec((NPAD, AUG), lambda: (0, 0)),
            pl.BlockSpec((NPAD, AUG), lambda: (0, 0)),
            pl.BlockSpec((NPAD, 16), lambda: (0, 0)),
            pl.BlockSpec((8, 128), lambda: (0, 0)),
            pl.BlockSpec((1, DOUT), lambda: (0, 0)),
        ],
        out_specs=pl.BlockSpec((N, DOUT), lambda: (0, 0)),
        out_shape=jax.ShapeDtypeStruct((N, DOUT), F32),
        compiler_params=pltpu.CompilerParams(
            vmem_limit_bytes=100 * 1024 * 1024),
    )(acc2, hga0, hga1, ad16, cpad, b2)


def kernel(nodes, edge_index, edge_weight, emb, W1, b1, W2, att_src, att_dst,
           lin_edge, att_edge, b2):
    del nodes
    src = edge_index[0].astype(I32)
    dst = edge_index[1].astype(I32)
    w = edge_weight.astype(F32)

    padn = EP - E
    srcp = jnp.concatenate([src, jnp.zeros((padn,), I32)])
    dstp = jnp.concatenate([dst, jnp.full((padn,), DUMMY, I32)])
    wp = jnp.concatenate([w, jnp.zeros((padn,), F32)])
    src2 = srcp.reshape(EP // 128, 128)
    dst2 = dstp.reshape(EP // 128, 128)
    w2 = wp.reshape(EP // 128, 128)
    srcb2 = jnp.concatenate([srcp, srcp + NPAD]).reshape(2 * EP // 128, 128)

    emb_pad = jnp.pad(emb, ((0, NPAD - N), (0, 0)))

    deg_tab = _sc_deg(dst2, w2)
    g2v = _tc1(emb_pad, W1, deg_tab)
    g2flat = g2v.reshape(2 * NPAD, 128)
    acc1 = _sc_acc1(srcb2, dst2, w2, g2flat)
    hga0, hga1, ad16, cpad = _tc2(acc1, g2v, deg_tab, b1[None, :], W2,
                                  att_src[None, :], att_dst[None, :],
                                  lin_edge, att_edge[None, :])
    hgab = jnp.concatenate([hga0, hga1])
    acc2 = _sc_acc2(srcb2, dst2, w2, hgab, ad16, cpad)
    return _tc3(acc2, hga0, hga1, ad16, cpad, b2[None, :])
